# baseline (device time: 171763 ns/iter reference)
import jax
import jax.numpy as jnp
from jax import lax
from jax.experimental import pallas as pl
from jax.experimental.pallas import tpu as pltpu

N_DEV = 16
CROWS = 512
FWD = N_DEV // 2
BWD = N_DEV - 1 - FWD


def _ag_body(w_ref, w_all_ref, fs, fr, bs, br):
    my = lax.axis_index("i")
    left = lax.rem(my + N_DEV - 1, N_DEV)
    right = lax.rem(my + 1, N_DEV)

    barrier = pltpu.get_barrier_semaphore()
    pl.semaphore_signal(barrier, inc=1, device_id=(left,),
                        device_id_type=pl.DeviceIdType.MESH)
    pl.semaphore_signal(barrier, inc=1, device_id=(right,),
                        device_id_type=pl.DeviceIdType.MESH)
    pl.semaphore_wait(barrier, 2)

    w_all_ref[pl.ds(my * CROWS, CROWS), :] = w_ref[...]

    def chunk(c):
        return w_all_ref.at[pl.ds(lax.rem(c + 2 * N_DEV, N_DEV) * CROWS, CROWS), :]

    sends = []
    for h in range(FWD):
        s = pltpu.make_async_remote_copy(
            src_ref=chunk(my - h), dst_ref=chunk(my - h),
            send_sem=fs.at[h], recv_sem=fr.at[h],
            device_id=(right,), device_id_type=pl.DeviceIdType.MESH)
        s.start()
        sends.append(s)
        if h < BWD:
            s = pltpu.make_async_remote_copy(
                src_ref=chunk(my + h), dst_ref=chunk(my + h),
                send_sem=bs.at[h], recv_sem=br.at[h],
                device_id=(left,), device_id_type=pl.DeviceIdType.MESH)
            s.start()
            sends.append(s)
        pltpu.make_async_remote_copy(
            src_ref=chunk(my - 1 - h), dst_ref=chunk(my - 1 - h),
            send_sem=fs.at[h], recv_sem=fr.at[h],
            device_id=(left,), device_id_type=pl.DeviceIdType.MESH).wait_recv()
        if h < BWD:
            pltpu.make_async_remote_copy(
                src_ref=chunk(my + 1 + h), dst_ref=chunk(my + 1 + h),
                send_sem=bs.at[h], recv_sem=br.at[h],
                device_id=(right,), device_id_type=pl.DeviceIdType.MESH).wait_recv()
    for s in sends:
        s.wait_send()


def _all_gather_weights(w_shard):
    return pl.pallas_call(
        _ag_body,
        out_shape=jax.ShapeDtypeStruct((N_DEV * CROWS, 512), jnp.bfloat16),
        in_specs=[pl.BlockSpec(memory_space=pltpu.VMEM)],
        out_specs=pl.BlockSpec(memory_space=pltpu.VMEM),
        scratch_shapes=[
            pltpu.SemaphoreType.DMA((FWD,)),
            pltpu.SemaphoreType.DMA((FWD,)),
            pltpu.SemaphoreType.DMA((BWD,)),
            pltpu.SemaphoreType.DMA((BWD,)),
        ],
        compiler_params=pltpu.CompilerParams(collective_id=0),
    )(w_shard)


def _attn_body(x_ref, wq_ref, wo_ref, kt_ref, vt_ref, out_ref):
    g = pl.program_id(1)
    row = lax.broadcasted_iota(jnp.int32, (128, 128), 0) // 64
    col = lax.broadcasted_iota(jnp.int32, (128, 128), 1) // 64
    mask = col <= row

    acc = jnp.zeros((128, 512), jnp.float32)
    for gg in range(4):
        wq_h = wq_ref[gg * 64:(gg + 1) * 64, :]
        q = lax.dot_general(x_ref[...], wq_h, (((1,), (1,)), ((), ())),
                            preferred_element_type=jnp.float32)
        q = q.astype(jnp.bfloat16)
        k = kt_ref[0, gg]
        s = lax.dot_general(q, k, (((1,), (1,)), ((), ())),
                            preferred_element_type=jnp.float32) * 0.125
        s = jnp.where(mask, s, -1e9)
        m = jnp.max(s, axis=-1, keepdims=True)
        p = jnp.exp(s - m)
        p = p / jnp.sum(p, axis=-1, keepdims=True)
        ctx = jnp.dot(p.astype(jnp.bfloat16), vt_ref[0, gg],
                      preferred_element_type=jnp.float32)
        acc = acc + jnp.dot(ctx.astype(jnp.bfloat16),
                            wo_ref[gg * 64:(gg + 1) * 64, :],
                            preferred_element_type=jnp.float32)

    @pl.when(g == 0)
    def _():
        out_ref[...] = acc

    @pl.when(g != 0)
    def _():
        out_ref[...] += acc


def _attention(x2d, w_all, kt, vt):
    return pl.pallas_call(
        _attn_body,
        grid=(2, N_DEV),
        out_shape=jax.ShapeDtypeStruct((256, 512), jnp.float32),
        in_specs=[
            pl.BlockSpec((128, 512), lambda b, g: (b, 0)),
            pl.BlockSpec((256, 512), lambda b, g: (2 * g, 0)),
            pl.BlockSpec((256, 512), lambda b, g: (2 * g + 1, 0)),
            pl.BlockSpec((1, 4, 128, 64), lambda b, g: (b, g, 0, 0)),
            pl.BlockSpec((1, 4, 128, 64), lambda b, g: (b, g, 0, 0)),
        ],
        out_specs=pl.BlockSpec((128, 512), lambda b, g: (b, 0)),
    )(x2d, w_all, w_all, kt, vt)


def kernel(x, Wq, K_ext, V_ext, Wo):
    bf16 = jnp.bfloat16
    my = lax.axis_index("i")

    x2d = x.reshape(256, 512).astype(bf16)
    w_shard = jnp.concatenate([Wq.astype(bf16).T, Wo.astype(bf16)], axis=0)
    b0 = my * 2
    kt = jnp.transpose(
        lax.dynamic_slice_in_dim(K_ext, b0, 2, 0), (0, 2, 1, 3)).astype(bf16)
    vt = jnp.transpose(
        lax.dynamic_slice_in_dim(V_ext, b0, 2, 0), (0, 2, 1, 3)).astype(bf16)

    w_all = _all_gather_weights(w_shard)
    out = _attention(x2d, w_all, kt, vt)
    return out.reshape(2, 128, 512)


# device time: 159718 ns/iter; 1.0754x vs baseline; 1.0754x over previous
import jax
import jax.numpy as jnp
from jax import lax
from jax.experimental import pallas as pl
from jax.experimental.pallas import tpu as pltpu

N_DEV = 16
CROWS = 512
FWD = N_DEV // 2
BWD = N_DEV - 1 - FWD


def _c2m(p):
    p = lax.rem(p + 2 * N_DEV, N_DEV)
    c = p // 4
    i = lax.rem(p, 4)
    z = jnp.where(lax.rem(c, 2) == 0, i, 3 - i)
    return 4 * z + c


def _ag_body(w_ref, w_all_ref, fs, fr, bs, br):
    my = lax.axis_index("i")
    q = lax.rem(my, 4)
    zz = my // 4
    my_p = 4 * q + jnp.where(lax.rem(q, 2) == 0, zz, 3 - zz)
    left = _c2m(my_p - 1)
    right = _c2m(my_p + 1)

    barrier = pltpu.get_barrier_semaphore()
    pl.semaphore_signal(barrier, inc=1, device_id=(left,),
                        device_id_type=pl.DeviceIdType.MESH)
    pl.semaphore_signal(barrier, inc=1, device_id=(right,),
                        device_id_type=pl.DeviceIdType.MESH)
    pl.semaphore_wait(barrier, 2)

    w_all_ref[pl.ds(my * CROWS, CROWS), :] = w_ref[...]

    def chunk(p):
        return w_all_ref.at[pl.ds(_c2m(p) * CROWS, CROWS), :]

    sends = []
    for h in range(FWD):
        s = pltpu.make_async_remote_copy(
            src_ref=chunk(my_p - h), dst_ref=chunk(my_p - h),
            send_sem=fs.at[h], recv_sem=fr.at[h],
            device_id=(right,), device_id_type=pl.DeviceIdType.MESH)
        s.start()
        sends.append(s)
        if h < BWD:
            s = pltpu.make_async_remote_copy(
                src_ref=chunk(my_p + h), dst_ref=chunk(my_p + h),
                send_sem=bs.at[h], recv_sem=br.at[h],
                device_id=(left,), device_id_type=pl.DeviceIdType.MESH)
            s.start()
            sends.append(s)
        pltpu.make_async_remote_copy(
            src_ref=chunk(my_p - 1 - h), dst_ref=chunk(my_p - 1 - h),
            send_sem=fs.at[h], recv_sem=fr.at[h],
            device_id=(left,), device_id_type=pl.DeviceIdType.MESH).wait_recv()
        if h < BWD:
            pltpu.make_async_remote_copy(
                src_ref=chunk(my_p + 1 + h), dst_ref=chunk(my_p + 1 + h),
                send_sem=bs.at[h], recv_sem=br.at[h],
                device_id=(right,), device_id_type=pl.DeviceIdType.MESH).wait_recv()
    for s in sends:
        s.wait_send()


def _all_gather_weights(w_shard):
    return pl.pallas_call(
        _ag_body,
        out_shape=jax.ShapeDtypeStruct((N_DEV * CROWS, 512), jnp.bfloat16),
        in_specs=[pl.BlockSpec(memory_space=pltpu.VMEM)],
        out_specs=pl.BlockSpec(memory_space=pltpu.VMEM),
        scratch_shapes=[
            pltpu.SemaphoreType.DMA((FWD,)),
            pltpu.SemaphoreType.DMA((FWD,)),
            pltpu.SemaphoreType.DMA((BWD,)),
            pltpu.SemaphoreType.DMA((BWD,)),
        ],
        compiler_params=pltpu.CompilerParams(collective_id=0),
    )(w_shard)


def _attn_body(x_ref, wq_ref, wo_ref, kt_ref, vt_ref, out_ref):
    g = pl.program_id(1)
    row = lax.broadcasted_iota(jnp.int32, (128, 128), 0) // 64
    col = lax.broadcasted_iota(jnp.int32, (128, 128), 1) // 64
    mask = col <= row

    acc = jnp.zeros((128, 512), jnp.float32)
    for gg in range(4):
        wq_h = wq_ref[gg * 64:(gg + 1) * 64, :]
        q = lax.dot_general(x_ref[...], wq_h, (((1,), (1,)), ((), ())),
                            preferred_element_type=jnp.float32)
        q = q.astype(jnp.bfloat16)
        k = kt_ref[0, gg]
        s = lax.dot_general(q, k, (((1,), (1,)), ((), ())),
                            preferred_element_type=jnp.float32) * 0.125
        s = jnp.where(mask, s, -1e9)
        m = jnp.max(s, axis=-1, keepdims=True)
        p = jnp.exp(s - m)
        p = p / jnp.sum(p, axis=-1, keepdims=True)
        ctx = jnp.dot(p.astype(jnp.bfloat16), vt_ref[0, gg],
                      preferred_element_type=jnp.float32)
        acc = acc + jnp.dot(ctx.astype(jnp.bfloat16),
                            wo_ref[gg * 64:(gg + 1) * 64, :],
                            preferred_element_type=jnp.float32)

    @pl.when(g == 0)
    def _():
        out_ref[...] = acc

    @pl.when(g != 0)
    def _():
        out_ref[...] += acc


def _attention(x2d, w_all, kt, vt):
    return pl.pallas_call(
        _attn_body,
        grid=(2, N_DEV),
        out_shape=jax.ShapeDtypeStruct((256, 512), jnp.float32),
        in_specs=[
            pl.BlockSpec((128, 512), lambda b, g: (b, 0)),
            pl.BlockSpec((256, 512), lambda b, g: (2 * g, 0)),
            pl.BlockSpec((256, 512), lambda b, g: (2 * g + 1, 0)),
            pl.BlockSpec((1, 4, 128, 64), lambda b, g: (b, g, 0, 0)),
            pl.BlockSpec((1, 4, 128, 64), lambda b, g: (b, g, 0, 0)),
        ],
        out_specs=pl.BlockSpec((128, 512), lambda b, g: (b, 0)),
    )(x2d, w_all, w_all, kt, vt)


def kernel(x, Wq, K_ext, V_ext, Wo):
    bf16 = jnp.bfloat16
    my = lax.axis_index("i")

    x2d = x.reshape(256, 512).astype(bf16)
    w_shard = jnp.concatenate([Wq.astype(bf16).T, Wo.astype(bf16)], axis=0)
    b0 = my * 2
    kt = jnp.transpose(
        lax.dynamic_slice_in_dim(K_ext, b0, 2, 0), (0, 2, 1, 3)).astype(bf16)
    vt = jnp.transpose(
        lax.dynamic_slice_in_dim(V_ext, b0, 2, 0), (0, 2, 1, 3)).astype(bf16)

    w_all = _all_gather_weights(w_shard)
    out = _attention(x2d, w_all, kt, vt)
    return out.reshape(2, 128, 512)


# device time: 97431 ns/iter; 1.7629x vs baseline; 1.6393x over previous
import jax
import jax.numpy as jnp
from jax import lax
from jax.experimental import pallas as pl
from jax.experimental.pallas import tpu as pltpu

N_DEV = 16
CROWS = 256
FWD = N_DEV // 2
BWD = N_DEV - 1 - FWD


def _c2m(p):
    p = lax.rem(p + 2 * N_DEV, N_DEV)
    c = p // 4
    i = lax.rem(p, 4)
    z = jnp.where(lax.rem(c, 2) == 0, i, 3 - i)
    return 4 * z + c


def _ag_body(x_ref, wqT_ref, wo_ref, q2d_ref, wo_all_ref, wqT_all_ref,
             fsq, frq, fso, fro, bsq, brq, bso, bro):
    my = lax.axis_index("i")
    qq = lax.rem(my, 4)
    zz = my // 4
    my_p = 4 * qq + jnp.where(lax.rem(qq, 2) == 0, zz, 3 - zz)
    left = _c2m(my_p - 1)
    right = _c2m(my_p + 1)

    barrier = pltpu.get_barrier_semaphore()
    pl.semaphore_signal(barrier, inc=1, device_id=(left,),
                        device_id_type=pl.DeviceIdType.MESH)
    pl.semaphore_signal(barrier, inc=1, device_id=(right,),
                        device_id_type=pl.DeviceIdType.MESH)
    pl.semaphore_wait(barrier, 2)

    wqT_all_ref[pl.ds(my * CROWS, CROWS), :] = wqT_ref[...]
    wo_all_ref[pl.ds(my * CROWS, CROWS), :] = wo_ref[...]

    def chunk(buf, p):
        return buf.at[pl.ds(_c2m(p) * CROWS, CROWS), :]

    def rdma(p, sem_pair, h, target):
        return [
            pltpu.make_async_remote_copy(
                src_ref=chunk(buf, p), dst_ref=chunk(buf, p),
                send_sem=ss.at[h], recv_sem=rs.at[h],
                device_id=(target,), device_id_type=pl.DeviceIdType.MESH)
            for buf, (ss, rs) in ((wqT_all_ref, sem_pair[0]),
                                  (wo_all_ref, sem_pair[1]))
        ]

    fwd_sems = ((fsq, frq), (fso, fro))
    bwd_sems = ((bsq, brq), (bso, bro))
    sends = []
    for h in range(FWD):
        for s in rdma(my_p - h, fwd_sems, h, right):
            s.start()
            sends.append(s)
        if h < BWD:
            for s in rdma(my_p + h, bwd_sems, h, left):
                s.start()
                sends.append(s)
        for r in rdma(my_p - 1 - h, fwd_sems, h, left):
            r.wait_recv()
        if h < BWD:
            for r in rdma(my_p + 1 + h, bwd_sems, h, right):
                r.wait_recv()
    for s in sends:
        s.wait_send()

    q2d_ref[...] = lax.dot_general(
        x_ref[...], wqT_all_ref[...], (((1,), (1,)), ((), ())),
        preferred_element_type=jnp.float32).astype(jnp.bfloat16)


def _ag_and_qproj(x2d, wqT, wo):
    return pl.pallas_call(
        _ag_body,
        out_shape=(
            jax.ShapeDtypeStruct((256, 4096), jnp.bfloat16),
            jax.ShapeDtypeStruct((N_DEV * CROWS, 512), jnp.bfloat16),
        ),
        in_specs=[pl.BlockSpec(memory_space=pltpu.VMEM)] * 3,
        out_specs=[pl.BlockSpec(memory_space=pltpu.VMEM)] * 2,
        scratch_shapes=[pltpu.VMEM((N_DEV * CROWS, 512), jnp.bfloat16)]
        + [pltpu.SemaphoreType.DMA((FWD,))] * 4
        + [pltpu.SemaphoreType.DMA((BWD,))] * 4,
        compiler_params=pltpu.CompilerParams(collective_id=0),
    )(x2d, wqT, wo)


def _attn_body(q_ref, k_ref, v_ref, o_ref):
    row = lax.broadcasted_iota(jnp.int32, (128, 128), 0) // 64
    col = lax.broadcasted_iota(jnp.int32, (128, 128), 1) // 64
    mask = (col <= row)[None]
    s = lax.dot_general(q_ref[...], k_ref[...], (((2,), (2,)), ((0,), (0,))),
                        preferred_element_type=jnp.float32) * 0.125
    s = jnp.where(mask, s, -1e9)
    m = jnp.max(s, axis=-1, keepdims=True)
    p = jnp.exp(s - m)
    p = p / jnp.sum(p, axis=-1, keepdims=True)
    o_ref[...] = lax.dot_general(
        p.astype(jnp.bfloat16), v_ref[...], (((2,), (1,)), ((0,), (0,))),
        preferred_element_type=jnp.float32).astype(jnp.bfloat16)


def _attention(q3, k3, v3):
    return pl.pallas_call(
        _attn_body,
        grid=(4,),
        out_shape=jax.ShapeDtypeStruct((128, 128, 64), jnp.bfloat16),
        in_specs=[pl.BlockSpec((32, 128, 64), lambda g: (g, 0, 0))] * 3,
        out_specs=pl.BlockSpec((32, 128, 64), lambda g: (g, 0, 0)),
    )(q3, k3, v3)


def _out_body(c_ref, w_ref, o_ref):
    o_ref[...] = jnp.dot(c_ref[...], w_ref[...],
                         preferred_element_type=jnp.float32)


def _out_proj(ctx2d, wo_all):
    return pl.pallas_call(
        _out_body,
        out_shape=jax.ShapeDtypeStruct((256, 512), jnp.float32),
        in_specs=[pl.BlockSpec(memory_space=pltpu.VMEM)] * 2,
        out_specs=pl.BlockSpec(memory_space=pltpu.VMEM),
    )(ctx2d, wo_all)


def kernel(x, Wq, K_ext, V_ext, Wo):
    bf16 = jnp.bfloat16
    my = lax.axis_index("i")

    x2d = x.reshape(256, 512).astype(bf16)
    wqT = Wq.astype(bf16).T
    wo = Wo.astype(bf16)
    b0 = my * 2
    k3 = jnp.transpose(
        lax.dynamic_slice_in_dim(K_ext, b0, 2, 0),
        (0, 2, 1, 3)).astype(bf16).reshape(128, 128, 64)
    v3 = jnp.transpose(
        lax.dynamic_slice_in_dim(V_ext, b0, 2, 0),
        (0, 2, 1, 3)).astype(bf16).reshape(128, 128, 64)

    q2d, wo_all = _ag_and_qproj(x2d, wqT, wo)
    q3 = q2d.reshape(2, 128, 64, 64).transpose(0, 2, 1, 3).reshape(128, 128, 64)
    ctx3 = _attention(q3, k3, v3)
    ctx2d = ctx3.reshape(2, 64, 128, 64).transpose(0, 2, 1, 3).reshape(256, 4096)
    out = _out_proj(ctx2d, wo_all)
    return out.reshape(2, 128, 512)


# device time: 77274 ns/iter; 2.2228x vs baseline; 1.2609x over previous
import jax
import jax.numpy as jnp
from jax import lax
from jax.experimental import pallas as pl
from jax.experimental.pallas import tpu as pltpu

N_DEV = 16
CROWS = 512
FWD = N_DEV // 2
BWD = N_DEV - 1 - FWD


def _c2m(p):
    p = lax.rem(p + 2 * N_DEV, N_DEV)
    c = p // 4
    i = lax.rem(p, 4)
    z = jnp.where(lax.rem(c, 2) == 0, i, 3 - i)
    return 4 * z + c


def _ag_body(x_ref, w_ref, s_ref, q2d_ref, w_all_ref, s_all_ref,
             fsw, frw, fss, frs, bsw, brw, bss, brs):
    my = lax.axis_index("i")
    qq = lax.rem(my, 4)
    zz = my // 4
    my_p = 4 * qq + jnp.where(lax.rem(qq, 2) == 0, zz, 3 - zz)
    left = _c2m(my_p - 1)
    right = _c2m(my_p + 1)

    barrier = pltpu.get_barrier_semaphore()
    pl.semaphore_signal(barrier, inc=1, device_id=(left,),
                        device_id_type=pl.DeviceIdType.MESH)
    pl.semaphore_signal(barrier, inc=1, device_id=(right,),
                        device_id_type=pl.DeviceIdType.MESH)
    pl.semaphore_wait(barrier, 2)

    w_all_ref[pl.ds(my * CROWS, CROWS), :] = w_ref[...]
    s_all_ref[pl.ds(my, 1), :] = s_ref[...]

    def rdma(p, sems, h, target):
        m = _c2m(p)
        out = []
        for buf, rows, (ss, rs) in ((w_all_ref, CROWS, sems[0]),
                                    (s_all_ref, 1, sems[1])):
            sl = buf.at[pl.ds(m * rows, rows), :]
            out.append(pltpu.make_async_remote_copy(
                src_ref=sl, dst_ref=sl, send_sem=ss.at[h], recv_sem=rs.at[h],
                device_id=(target,), device_id_type=pl.DeviceIdType.MESH))
        return out

    fwd_sems = ((fsw, frw), (fss, frs))
    bwd_sems = ((bsw, brw), (bss, brs))
    sends = []
    for h in range(FWD):
        for s in rdma(my_p - h, fwd_sems, h, right):
            s.start()
            sends.append(s)
        if h < BWD:
            for s in rdma(my_p + h, bwd_sems, h, left):
                s.start()
                sends.append(s)
        for r in rdma(my_p - 1 - h, fwd_sems, h, left):
            r.wait_recv()
        if h < BWD:
            for r in rdma(my_p + 1 + h, bwd_sems, h, right):
                r.wait_recv()
    for s in sends:
        s.wait_send()

    x = x_ref[...]
    for c in range(N_DEV):
        wq_c = w_all_ref[CROWS * c:CROWS * c + 256, :].astype(jnp.bfloat16)
        q_c = lax.dot_general(x, wq_c, (((1,), (1,)), ((), ())),
                              preferred_element_type=jnp.float32)
        q_c = q_c * s_all_ref[c:c + 1, 0:256]
        q2d_ref[:, 256 * c:256 * c + 256] = q_c.astype(jnp.bfloat16)


def _ag_and_qproj(x2d, w_i8, s_shard):
    return pl.pallas_call(
        _ag_body,
        out_shape=(
            jax.ShapeDtypeStruct((256, 4096), jnp.bfloat16),
            jax.ShapeDtypeStruct((N_DEV * CROWS, 512), jnp.int8),
            jax.ShapeDtypeStruct((N_DEV, 512), jnp.float32),
        ),
        in_specs=[pl.BlockSpec(memory_space=pltpu.VMEM)] * 3,
        out_specs=[pl.BlockSpec(memory_space=pltpu.VMEM)] * 3,
        scratch_shapes=[pltpu.SemaphoreType.DMA((FWD,))] * 4
        + [pltpu.SemaphoreType.DMA((BWD,))] * 4,
        compiler_params=pltpu.CompilerParams(collective_id=0),
    )(x2d, w_i8, s_shard)


def _attn_body(q_ref, k_ref, v_ref, o_ref):
    row = lax.broadcasted_iota(jnp.int32, (128, 128), 0) // 64
    col = lax.broadcasted_iota(jnp.int32, (128, 128), 1) // 64
    mask = (col <= row)[None]
    s = lax.dot_general(q_ref[...], k_ref[...], (((2,), (2,)), ((0,), (0,))),
                        preferred_element_type=jnp.float32) * 0.125
    s = jnp.where(mask, s, -1e9)
    m = jnp.max(s, axis=-1, keepdims=True)
    p = jnp.exp(s - m)
    p = p / jnp.sum(p, axis=-1, keepdims=True)
    o_ref[...] = lax.dot_general(
        p.astype(jnp.bfloat16), v_ref[...], (((2,), (1,)), ((0,), (0,))),
        preferred_element_type=jnp.float32).astype(jnp.bfloat16)


def _attention(q3, k3, v3):
    return pl.pallas_call(
        _attn_body,
        grid=(4,),
        out_shape=jax.ShapeDtypeStruct((128, 128, 64), jnp.bfloat16),
        in_specs=[pl.BlockSpec((32, 128, 64), lambda g: (g, 0, 0))] * 3,
        out_specs=pl.BlockSpec((32, 128, 64), lambda g: (g, 0, 0)),
    )(q3, k3, v3)


def _out_body(c_ref, w_all_ref, s_ref, o_ref):
    acc = jnp.zeros((256, 512), jnp.float32)
    for c in range(N_DEV):
        ctx_c = c_ref[:, 256 * c:256 * c + 256] * s_ref[c:c + 1, 256:512]
        wo_c = w_all_ref[CROWS * c + 256:CROWS * c + 512, :].astype(jnp.bfloat16)
        acc = acc + jnp.dot(ctx_c.astype(jnp.bfloat16), wo_c,
                            preferred_element_type=jnp.float32)
    o_ref[...] = acc


def _out_proj(ctx2d, w_all, s_all):
    return pl.pallas_call(
        _out_body,
        out_shape=jax.ShapeDtypeStruct((256, 512), jnp.float32),
        in_specs=[pl.BlockSpec(memory_space=pltpu.VMEM)] * 3,
        out_specs=pl.BlockSpec(memory_space=pltpu.VMEM),
    )(ctx2d, w_all, s_all)


def kernel(x, Wq, K_ext, V_ext, Wo):
    bf16 = jnp.bfloat16
    my = lax.axis_index("i")

    x2d = x.reshape(256, 512).astype(bf16)
    wqT = Wq.T
    wo = Wo
    sq = jnp.maximum(jnp.max(jnp.abs(wqT), axis=1), 1e-20) / 127.0
    so = jnp.maximum(jnp.max(jnp.abs(wo), axis=1), 1e-20) / 127.0
    wq_i8 = jnp.round(wqT / sq[:, None]).astype(jnp.int8)
    wo_i8 = jnp.round(wo / so[:, None]).astype(jnp.int8)
    w_i8 = jnp.concatenate([wq_i8, wo_i8], axis=0)
    s_shard = jnp.concatenate([sq, so]).reshape(1, 512).astype(jnp.float32)

    b0 = my * 2
    k3 = jnp.transpose(
        lax.dynamic_slice_in_dim(K_ext, b0, 2, 0),
        (0, 2, 1, 3)).astype(bf16).reshape(128, 128, 64)
    v3 = jnp.transpose(
        lax.dynamic_slice_in_dim(V_ext, b0, 2, 0),
        (0, 2, 1, 3)).astype(bf16).reshape(128, 128, 64)

    q2d, w_all, s_all = _ag_and_qproj(x2d, w_i8, s_shard)
    q3 = q2d.reshape(2, 128, 64, 64).transpose(0, 2, 1, 3).reshape(128, 128, 64)
    ctx3 = _attention(q3, k3, v3)
    ctx2d = ctx3.reshape(2, 64, 128, 64).transpose(0, 2, 1, 3).reshape(256, 4096)
    out = _out_proj(ctx2d, w_all, s_all)
    return out.reshape(2, 128, 512)


# device time: 67994 ns/iter; 2.5261x vs baseline; 1.1365x over previous
import jax
import jax.numpy as jnp
from jax import lax
from jax.experimental import pallas as pl
from jax.experimental.pallas import tpu as pltpu

N_DEV = 16
CROWS = 512
FWD = N_DEV // 2
BWD = N_DEV - 1 - FWD


def _c2m(p):
    p = lax.rem(p + 2 * N_DEV, N_DEV)
    c = p // 4
    i = lax.rem(p, 4)
    z = jnp.where(lax.rem(c, 2) == 0, i, 3 - i)
    return 4 * z + c


def _ag_body(x_ref, w_ref, s_ref, q2d_ref, w_all_ref, s_all_ref,
             fsw, frw, fss, frs, bsw, brw, bss, brs):
    my = lax.axis_index("i")
    qq = lax.rem(my, 4)
    zz = my // 4
    my_p = 4 * qq + jnp.where(lax.rem(qq, 2) == 0, zz, 3 - zz)
    left = _c2m(my_p - 1)
    right = _c2m(my_p + 1)

    barrier = pltpu.get_barrier_semaphore()
    pl.semaphore_signal(barrier, inc=1, device_id=(left,),
                        device_id_type=pl.DeviceIdType.MESH)
    pl.semaphore_signal(barrier, inc=1, device_id=(right,),
                        device_id_type=pl.DeviceIdType.MESH)
    pl.semaphore_wait(barrier, 2)

    w_all_ref[pl.ds(my * CROWS, CROWS), :] = w_ref[...]
    s_all_ref[pl.ds(my, 1), :] = s_ref[...]

    HALF = CROWS // 2

    def w_rdma(p, ss, rs, h, sub, target):
        sl = w_all_ref.at[pl.ds(_c2m(p) * CROWS + sub * HALF, HALF), :]
        return pltpu.make_async_remote_copy(
            src_ref=sl, dst_ref=sl, send_sem=ss.at[h, sub],
            recv_sem=rs.at[h, sub],
            device_id=(target,), device_id_type=pl.DeviceIdType.MESH)

    def s_rdma(p, ss, rs, h, target):
        sl = s_all_ref.at[pl.ds(_c2m(p), 1), :]
        return pltpu.make_async_remote_copy(
            src_ref=sl, dst_ref=sl, send_sem=ss.at[h], recv_sem=rs.at[h],
            device_id=(target,), device_id_type=pl.DeviceIdType.MESH)

    sends = []
    for h in range(FWD):
        for sub in (0, 1):
            if h > 0:
                w_rdma(my_p - h, fsw, frw, h - 1, sub, left).wait_recv()
            s = w_rdma(my_p - h, fsw, frw, h, sub, right)
            s.start()
            sends.append(s)
            if h < BWD:
                if h > 0:
                    w_rdma(my_p + h, bsw, brw, h - 1, sub, right).wait_recv()
                s = w_rdma(my_p + h, bsw, brw, h, sub, left)
                s.start()
                sends.append(s)
        if h > 0:
            s_rdma(my_p - h, fss, frs, h - 1, left).wait_recv()
        s = s_rdma(my_p - h, fss, frs, h, right)
        s.start()
        sends.append(s)
        if h < BWD:
            if h > 0:
                s_rdma(my_p + h, bss, brs, h - 1, right).wait_recv()
            s = s_rdma(my_p + h, bss, brs, h, left)
            s.start()
            sends.append(s)
    for sub in (0, 1):
        w_rdma(my_p - FWD, fsw, frw, FWD - 1, sub, left).wait_recv()
        w_rdma(my_p + BWD, bsw, brw, BWD - 1, sub, right).wait_recv()
    s_rdma(my_p - FWD, fss, frs, FWD - 1, left).wait_recv()
    s_rdma(my_p + BWD, bss, brs, BWD - 1, right).wait_recv()
    for s in sends:
        s.wait_send()

    x = x_ref[...]
    for c in range(N_DEV):
        wq_c = w_all_ref[CROWS * c:CROWS * c + 256, :].astype(jnp.bfloat16)
        q_c = lax.dot_general(x, wq_c, (((1,), (1,)), ((), ())),
                              preferred_element_type=jnp.float32)
        q_c = q_c * s_all_ref[c:c + 1, 0:256]
        q2d_ref[:, 256 * c:256 * c + 256] = q_c.astype(jnp.bfloat16)


def _ag_and_qproj(x2d, w_i8, s_shard):
    return pl.pallas_call(
        _ag_body,
        out_shape=(
            jax.ShapeDtypeStruct((256, 4096), jnp.bfloat16),
            jax.ShapeDtypeStruct((N_DEV * CROWS, 512), jnp.int8),
            jax.ShapeDtypeStruct((N_DEV, 512), jnp.float32),
        ),
        in_specs=[pl.BlockSpec(memory_space=pltpu.VMEM)] * 3,
        out_specs=[pl.BlockSpec(memory_space=pltpu.VMEM)] * 3,
        scratch_shapes=[
            pltpu.SemaphoreType.DMA((FWD, 2)),
            pltpu.SemaphoreType.DMA((FWD, 2)),
            pltpu.SemaphoreType.DMA((FWD,)),
            pltpu.SemaphoreType.DMA((FWD,)),
            pltpu.SemaphoreType.DMA((BWD, 2)),
            pltpu.SemaphoreType.DMA((BWD, 2)),
            pltpu.SemaphoreType.DMA((BWD,)),
            pltpu.SemaphoreType.DMA((BWD,)),
        ],
        compiler_params=pltpu.CompilerParams(collective_id=0),
    )(x2d, w_i8, s_shard)


def _attn_body(q_ref, k_ref, v_ref, o_ref):
    dims = (((2,), (2,)), ((0,), (0,)))

    def softmax_ctx(q, k, v):
        s = lax.dot_general(q, k, dims,
                            preferred_element_type=jnp.float32) * 0.125
        m = jnp.max(s, axis=-1, keepdims=True)
        p = jnp.exp(s - m)
        p = p / jnp.sum(p, axis=-1, keepdims=True)
        return lax.dot_general(
            p.astype(jnp.bfloat16), v, (((2,), (1,)), ((0,), (0,))),
            preferred_element_type=jnp.float32).astype(jnp.bfloat16)

    o_ref[:, 0:64, :] = softmax_ctx(
        q_ref[:, 0:64, :], k_ref[:, 0:64, :], v_ref[:, 0:64, :])
    o_ref[:, 64:128, :] = softmax_ctx(
        q_ref[:, 64:128, :], k_ref[...], v_ref[...])


def _attention(q3, k3, v3):
    return pl.pallas_call(
        _attn_body,
        grid=(4,),
        out_shape=jax.ShapeDtypeStruct((128, 128, 64), jnp.bfloat16),
        in_specs=[pl.BlockSpec((32, 128, 64), lambda g: (g, 0, 0))] * 3,
        out_specs=pl.BlockSpec((32, 128, 64), lambda g: (g, 0, 0)),
    )(q3, k3, v3)


def _out_body(c_ref, w_all_ref, s_ref, o_ref):
    acc = jnp.zeros((256, 512), jnp.float32)
    for c in range(N_DEV):
        ctx_c = c_ref[:, 256 * c:256 * c + 256] * s_ref[c:c + 1, 256:512]
        wo_c = w_all_ref[CROWS * c + 256:CROWS * c + 512, :].astype(jnp.bfloat16)
        acc = acc + jnp.dot(ctx_c.astype(jnp.bfloat16), wo_c,
                            preferred_element_type=jnp.float32)
    o_ref[...] = acc


def _out_proj(ctx2d, w_all, s_all):
    return pl.pallas_call(
        _out_body,
        out_shape=jax.ShapeDtypeStruct((256, 512), jnp.float32),
        in_specs=[pl.BlockSpec(memory_space=pltpu.VMEM)] * 3,
        out_specs=pl.BlockSpec(memory_space=pltpu.VMEM),
    )(ctx2d, w_all, s_all)


def kernel(x, Wq, K_ext, V_ext, Wo):
    bf16 = jnp.bfloat16
    my = lax.axis_index("i")

    x2d = x.reshape(256, 512).astype(bf16)
    wqT = Wq.T
    wo = Wo
    sq = jnp.maximum(jnp.max(jnp.abs(wqT), axis=1), 1e-20) / 127.0
    so = jnp.maximum(jnp.max(jnp.abs(wo), axis=1), 1e-20) / 127.0
    wq_i8 = jnp.round(wqT / sq[:, None]).astype(jnp.int8)
    wo_i8 = jnp.round(wo / so[:, None]).astype(jnp.int8)
    w_i8 = jnp.concatenate([wq_i8, wo_i8], axis=0)
    s_shard = jnp.concatenate([sq, so]).reshape(1, 512).astype(jnp.float32)

    b0 = my * 2
    k3 = jnp.transpose(
        lax.dynamic_slice_in_dim(K_ext, b0, 2, 0),
        (0, 2, 1, 3)).astype(bf16).reshape(128, 128, 64)
    v3 = jnp.transpose(
        lax.dynamic_slice_in_dim(V_ext, b0, 2, 0),
        (0, 2, 1, 3)).astype(bf16).reshape(128, 128, 64)

    q2d, w_all, s_all = _ag_and_qproj(x2d, w_i8, s_shard)
    q3 = q2d.reshape(2, 128, 64, 64).transpose(0, 2, 1, 3).reshape(128, 128, 64)
    ctx3 = _attention(q3, k3, v3)
    ctx2d = ctx3.reshape(2, 64, 128, 64).transpose(0, 2, 1, 3).reshape(256, 4096)
    out = _out_proj(ctx2d, w_all, s_all)
    return out.reshape(2, 128, 512)
